# baseline (speedup 1.0000x reference)
import functools

import jax
import jax.numpy as jnp
from jax import lax
from jax.experimental import pallas as pl
from jax.experimental.pallas import tpu as pltpu
from jax.experimental.pallas import tpu_sc as plsc

DMODEL = 1024
VOCAB = 100000
BATCH = 4
SEQLEN = 2048
TOTAL = BATCH * SEQLEN
NUM_WORKERS = 32
L_PER_W = SEQLEN // NUM_WORKERS
CHUNK = 32
M_PER_B = L_PER_W // CHUNK
NCHUNKS = BATCH * M_PER_B
NVEC = DMODEL // 16
NBUF = 2


def _position_embedding():
    pos = jnp.arange(SEQLEN, dtype=jnp.float32)[:, None]
    i = jnp.arange(DMODEL, dtype=jnp.float32)[None, :]
    inv_freq = 1.0 / jnp.power(10000.0, 2.0 * i / DMODEL)
    ang = pos * inv_freq
    dim_idx = jnp.arange(DMODEL)
    pe = jnp.where((dim_idx % 2 == 0)[None, :], jnp.sin(ang), jnp.cos(ang))
    return pe.astype(jnp.float32)


@functools.partial(
    pl.kernel,
    out_type=jax.ShapeDtypeStruct((TOTAL, DMODEL), jnp.float32),
    mesh=plsc.VectorSubcoreMesh(core_axis_name="c", subcore_axis_name="s"),
    scratch_types=[
        pltpu.VMEM((BATCH * L_PER_W,), jnp.int32),
        pltpu.VMEM((CHUNK, DMODEL), jnp.float32),
    ]
    + [pltpu.VMEM((CHUNK, DMODEL), jnp.float32) for _ in range(NBUF)]
    + [pltpu.SemaphoreType.DMA for _ in range(2 * NBUF + 1)],
)
def _sc_embed(ids_hbm, pe_hbm, table_hbm, out_hbm, idx_v, pe_v, *rest):
    bufs = rest[:NBUF]
    gsems = rest[NBUF:2 * NBUF]
    osems = rest[2 * NBUF:3 * NBUF]
    pe_sem = rest[3 * NBUF]

    wid = lax.axis_index("s") * 2 + lax.axis_index("c")
    l0 = wid * L_PER_W

    pltpu.sync_copy(ids_hbm.at[pl.ds(wid * BATCH * L_PER_W, BATCH * L_PER_W)],
                    idx_v)
    pe_cp = pltpu.async_copy(pe_hbm.at[pl.ds(l0, CHUNK)], pe_v, pe_sem)

    def gather(c):
        m, b = c // BATCH, c % BATCH
        return pltpu.async_copy(
            table_hbm.at[idx_v.at[pl.ds(b * L_PER_W + m * CHUNK, CHUNK)]],
            bufs[c % NBUF], gsems[c % NBUF],
        )

    gh = [None] * NBUF
    oh = [None] * NBUF
    for c in range(NBUF):
        gh[c] = gather(c)
    pe_cp.wait()

    for c in range(NCHUNKS):
        i = c % NBUF
        m, b = c // BATCH, c % BATCH
        gh[i].wait()

        nxt = c - 1 + NBUF
        if c >= 1 and nxt < NCHUNKS:
            j = (c - 1) % NBUF
            oh[j].wait()
            gh[j] = gather(nxt)

        if c == BATCH:
            pltpu.async_copy(pe_hbm.at[pl.ds(l0 + CHUNK, CHUNK)], pe_v,
                             pe_sem).wait()

        @plsc.parallel_loop(0, CHUNK * NVEC, unroll=8)
        def add_pe(k, _buf=bufs[i]):
            r = k >> 6
            off = (k & 63) * 16
            plsc.addupdate(_buf.at[r, pl.ds(off, 16)],
                           pe_v[r, pl.ds(off, 16)])

        out_base = b * SEQLEN + l0 + m * CHUNK
        oh[i] = pltpu.async_copy(bufs[i], out_hbm.at[pl.ds(out_base, CHUNK)],
                                 osems[i])
    for i in range(NBUF):
        oh[i].wait()


def kernel(seq, table):
    pe = _position_embedding()
    flat_ids = (seq.astype(jnp.int32)
                .reshape(BATCH, NUM_WORKERS, L_PER_W)
                .transpose(1, 0, 2)
                .reshape(TOTAL))
    out = _sc_embed(flat_ids, pe, table)
    return out.reshape(BATCH, SEQLEN, DMODEL)


# speedup vs baseline: 1.0443x; 1.0443x over previous
import functools

import jax
import jax.numpy as jnp
from jax import lax
from jax.experimental import pallas as pl
from jax.experimental.pallas import tpu as pltpu
from jax.experimental.pallas import tpu_sc as plsc

DMODEL = 1024
VOCAB = 100000
BATCH = 4
SEQLEN = 2048
TOTAL = BATCH * SEQLEN
NUM_WORKERS = 32
L_PER_W = SEQLEN // NUM_WORKERS
CHUNK = 32
HALF = CHUNK // 2
M_PER_B = L_PER_W // CHUNK
NCHUNKS = BATCH * M_PER_B
NVEC = DMODEL // 16
NBUF = 2


def _position_embedding():
    pos = jnp.arange(SEQLEN, dtype=jnp.float32)[:, None]
    i = jnp.arange(DMODEL, dtype=jnp.float32)[None, :]
    inv_freq = 1.0 / jnp.power(10000.0, 2.0 * i / DMODEL)
    ang = pos * inv_freq
    dim_idx = jnp.arange(DMODEL)
    pe = jnp.where((dim_idx % 2 == 0)[None, :], jnp.sin(ang), jnp.cos(ang))
    return pe.astype(jnp.float32)


@functools.partial(
    pl.kernel,
    out_type=jax.ShapeDtypeStruct((TOTAL, DMODEL), jnp.float32),
    mesh=plsc.VectorSubcoreMesh(core_axis_name="c", subcore_axis_name="s"),
    scratch_types=[
        pltpu.VMEM((BATCH * L_PER_W,), jnp.int32),
        pltpu.VMEM((HALF, DMODEL), jnp.float32),
        pltpu.VMEM((HALF, DMODEL), jnp.float32),
    ]
    + [pltpu.VMEM((CHUNK, DMODEL), jnp.float32) for _ in range(NBUF)]
    + [pltpu.SemaphoreType.DMA for _ in range(2 * NBUF + 2)],
)
def _sc_embed(ids_hbm, pe_hbm, table_hbm, out_hbm, idx_v, pe0, pe1, *rest):
    bufs = rest[:NBUF]
    gsems = rest[NBUF:2 * NBUF]
    osems = rest[2 * NBUF:3 * NBUF]
    pe_sems = rest[3 * NBUF:3 * NBUF + 2]
    pe_bufs = (pe0, pe1)

    wid = lax.axis_index("s") * 2 + lax.axis_index("c")
    l0 = wid * L_PER_W

    pltpu.sync_copy(ids_hbm.at[pl.ds(wid * BATCH * L_PER_W, BATCH * L_PER_W)],
                    idx_v)

    def load_pe(m):
        return [
            pltpu.async_copy(
                pe_hbm.at[pl.ds(l0 + m * CHUNK + h * HALF, HALF)],
                pe_bufs[h], pe_sems[h])
            for h in range(2)
        ]

    pe_cps = load_pe(0)

    def gather(c):
        m, b = c // BATCH, c % BATCH
        return pltpu.async_copy(
            table_hbm.at[idx_v.at[pl.ds(b * L_PER_W + m * CHUNK, CHUNK)]],
            bufs[c % NBUF], gsems[c % NBUF],
        )

    gh = [None] * NBUF
    oh = [[None, None] for _ in range(NBUF)]
    for c in range(NBUF):
        gh[c] = gather(c)
    pe_cps[0].wait()
    pe_cps[1].wait()

    for c in range(NCHUNKS):
        i = c % NBUF
        m, b = c // BATCH, c % BATCH
        gh[i].wait()

        nxt = c - 1 + NBUF
        if c >= 1 and nxt < NCHUNKS:
            j = (c - 1) % NBUF
            oh[j][0].wait()
            oh[j][1].wait()
            gh[j] = gather(nxt)

        if c == BATCH:
            pe_cps[0].wait()
            pe_cps[1].wait()

        out_base = b * SEQLEN + l0 + m * CHUNK
        for h in range(2):

            @plsc.parallel_loop(0, HALF * NVEC, unroll=16)
            def add_pe(k, _buf=bufs[i], _pe=pe_bufs[h], _h=h):
                r = k >> 6
                off = (k & 63) * 16
                plsc.addupdate(_buf.at[_h * HALF + r, pl.ds(off, 16)],
                               _pe[r, pl.ds(off, 16)])

            oh[i][h] = pltpu.async_copy(
                bufs[i].at[pl.ds(h * HALF, HALF)],
                out_hbm.at[pl.ds(out_base + h * HALF, HALF)], osems[i])

        if c == BATCH - 1:
            pe_cps = load_pe(1)

    for i in range(NBUF):
        oh[i][0].wait()
        oh[i][1].wait()


def kernel(seq, table):
    pe = _position_embedding()
    flat_ids = (seq.astype(jnp.int32)
                .reshape(BATCH, NUM_WORKERS, L_PER_W)
                .transpose(1, 0, 2)
                .reshape(TOTAL))
    out = _sc_embed(flat_ids, pe, table)
    return out.reshape(BATCH, SEQLEN, DMODEL)
